# baseline (speedup 1.0000x reference)
import functools

import jax
import jax.numpy as jnp
from jax import lax
from jax.experimental import pallas as pl
from jax.experimental.pallas import tpu as pltpu
from jax.experimental.pallas import tpu_sc as plsc

E = 320000
N = 10000
H = 128
R = 6
OUT = 128

NC = 2
NS = 16
NW = NC * NS
EPW = E // NW
EB = 80
NCHUNK = EPW // EB
NG = EB // 16
RPT = 624
ZC = 16
LANES = 16


def _sc_edge_scatter(x, rbft, idx, wt):
    mesh = plsc.VectorSubcoreMesh(core_axis_name="c", subcore_axis_name="s")

    @functools.partial(
        pl.kernel,
        out_type=jax.ShapeDtypeStruct((NC, N, H), jnp.float32),
        mesh=mesh,
        scratch_types=[
            pltpu.VMEM((EB, H), jnp.float32),
            pltpu.VMEM((EB, H), jnp.float32),
            pltpu.VMEM((EB, H), jnp.float32),
            pltpu.VMEM((EB, H), jnp.float32),
            pltpu.VMEM((R, EB), jnp.float32),
            pltpu.VMEM((R, EB), jnp.float32),
            pltpu.VMEM((EB,), jnp.int32),
            pltpu.VMEM((EB,), jnp.int32),
            pltpu.VMEM((R, H), jnp.float32),
            pltpu.VMEM_SHARED((N, H), jnp.float32),
            pltpu.SemaphoreType.DMA,
            pltpu.SemaphoreType.DMA,
            pltpu.SemaphoreType.DMA,
        ],
    )
    def body(x_hbm, rbft_hbm, i_hbm, wt_hbm, out_hbm,
             xb0, xb1, hb0, hb1, rb0, rb1, ib0, ib1,
             wtb, acc, lsem0, lsem1, zsem):
        cid = lax.axis_index("c")
        sid = lax.axis_index("s")
        wid = sid * NC + cid
        ebase = wid * EPW

        pltpu.async_copy(wt_hbm, wtb, lsem0).wait()

        def zrow(r2, _):
            for k in range(H // LANES):
                hb0[r2, pl.ds(k * LANES, LANES)] = jnp.zeros((LANES,), jnp.float32)
            return 0
        lax.fori_loop(0, EB, zrow, 0)
        row0 = sid * RPT

        def acc_phase(dst_of):
            descs = []
            for t in range(RPT // EB):
                descs.append(dst_of(row0 + t * EB, EB))
            descs.append(dst_of(row0 + (RPT // EB) * EB, RPT % EB))
            for src, dst in descs:
                pltpu.async_copy(src, dst, zsem)

            @pl.when(sid == NS - 1)
            def _():
                s2, d2 = dst_of(NS * RPT, N - NS * RPT)
                pltpu.async_copy(s2, d2, zsem).wait()
            for src, dst in descs:
                pltpu.make_async_copy(src, dst, zsem).wait()

        acc_phase(lambda r0, n: (hb0.at[pl.ds(0, n)], acc.at[pl.ds(r0, n)]))
        plsc.subcore_barrier()

        def start_load(c, xb, rb, ib, sem):
            e0 = ebase + c * EB
            pltpu.async_copy(x_hbm.at[pl.ds(e0, EB), :], xb, sem)
            for r in range(R):
                pltpu.async_copy(rbft_hbm.at[pl.ds(r * E + e0, EB)], rb.at[r], sem)
            pltpu.async_copy(i_hbm.at[pl.ds(e0, EB)], ib, sem)

        def wait_load(xb, rb, ib, sem):
            pltpu.make_async_copy(x_hbm.at[pl.ds(ebase, EB), :], xb, sem).wait()
            for r in range(R):
                pltpu.make_async_copy(rbft_hbm.at[pl.ds(r * E, EB)], rb.at[r], sem).wait()
            pltpu.make_async_copy(i_hbm.at[pl.ds(ebase, EB)], ib, sem).wait()

        splats = [jnp.full((LANES, 1), j, jnp.int32) for j in range(LANES)]
        _gd = lax.GatherDimensionNumbers(
            offset_dims=(), collapsed_slice_dims=(0,), start_index_map=(0,))

        def bcast(v, j):
            return lax.gather(v, splats[j], _gd, (1,),
                              mode=lax.GatherScatterMode.PROMISE_IN_BOUNDS)

        def compute_chunk(xb, rb, hb):
            for half in range(2):
                wtv = [[wtb[r, pl.ds((half * 4 + k) * LANES, LANES)]
                        for k in range(4)] for r in range(R)]

                def grp(g, _):
                    rv = [rb[r, pl.ds(g * LANES, LANES)] for r in range(R)]
                    for j in range(LANES):
                        row = g * LANES + j
                        cs = [bcast(rv[r], j) for r in range(R)]
                        for k in range(4):
                            kk = half * 4 + k
                            p0 = cs[0] * wtv[0][k] + cs[1] * wtv[1][k]
                            p1 = cs[2] * wtv[2][k] + cs[3] * wtv[3][k]
                            p2 = cs[4] * wtv[4][k] + cs[5] * wtv[5][k]
                            w = (p0 + p1) + p2
                            xv = xb[row, pl.ds(kk * LANES, LANES)]
                            hb[row, pl.ds(kk * LANES, LANES)] = w * xv
                    return 0
                lax.fori_loop(0, NG, grp, 0)

        def do_chunk(xb, rb, ib, hb, lsem, next_c, xbn, rbn, ibn, lsemn):
            wait_load(xb, rb, ib, lsem)
            start_load(next_c, xbn, rbn, ibn, lsemn)
            compute_chunk(xb, rb, hb)
            pltpu.sync_copy(hb, acc.at[ib], add=True)

        start_load(0, xb0, rb0, ib0, lsem0)

        def pair(it, _):
            c0 = it * 2
            do_chunk(xb0, rb0, ib0, hb0, lsem0, c0 + 1, xb1, rb1, ib1, lsem1)
            do_chunk(xb1, rb1, ib1, hb1, lsem1, c0 + 2, xb0, rb0, ib0, lsem0)
            return 0
        lax.fori_loop(0, (NCHUNK - 1) // 2, pair, 0)

        wait_load(xb0, rb0, ib0, lsem0)
        compute_chunk(xb0, rb0, hb0)
        pltpu.sync_copy(hb0, acc.at[ib0], add=True)

        plsc.subcore_barrier()

        acc_phase(lambda r0, n: (acc.at[pl.ds(r0, n)],
                                 out_hbm.at[cid, pl.ds(r0, n), :]))

    return body(x, rbft, idx, wt)


BR = 1000


def _mlp(parts, w1, b1, w2, b2, w3, b3, wout):
    def body(p_ref, w1_ref, b1_ref, w2_ref, b2_ref, w3_ref, b3_ref, wo_ref,
             o_ref):
        h = p_ref[0] + p_ref[1]

        def ff(h, w_ref, b_ref):
            y = lax.dot_general(h, w_ref[...], (((1,), (1,)), ((), ())),
                                precision=lax.Precision.HIGHEST,
                                preferred_element_type=jnp.float32)
            y = y + b_ref[...]
            return y * jax.nn.sigmoid(y)

        h = ff(h, w1_ref, b1_ref)
        h = ff(h, w2_ref, b2_ref)
        h = ff(h, w3_ref, b3_ref)
        o_ref[...] = lax.dot_general(h, wo_ref[...], (((1,), (1,)), ((), ())),
                                     precision=lax.Precision.HIGHEST,
                                     preferred_element_type=jnp.float32)

    wspec = pl.BlockSpec((H, H), lambda b: (0, 0))
    bspec = pl.BlockSpec((1, H), lambda b: (0, 0))
    return pl.pallas_call(
        body,
        grid=(N // BR,),
        in_specs=[
            pl.BlockSpec((NC, BR, H), lambda b: (0, b, 0)),
            wspec, bspec, wspec, bspec, wspec, bspec,
            pl.BlockSpec((OUT, H), lambda b: (0, 0)),
        ],
        out_specs=pl.BlockSpec((BR, OUT), lambda b: (b, 0)),
        out_shape=jax.ShapeDtypeStruct((N, OUT), jnp.float32),
    )(parts, w1, b1, w2, b2, w3, b3, wout)


def kernel(x, rbf, i, num_nodes, W_rbf, W1, b1, W2, b2, W3, b3, W_out):
    del num_nodes
    rbft = rbf.T.reshape(-1)
    idx = i.astype(jnp.int32)
    wt = W_rbf.T
    parts = _sc_edge_scatter(x, rbft, idx, wt)
    return _mlp(parts, W1, b1.reshape(1, H), W2, b2.reshape(1, H),
                W3, b3.reshape(1, H), W_out)


# speedup vs baseline: 1.0122x; 1.0122x over previous
import functools

import jax
import jax.numpy as jnp
from jax import lax
from jax.experimental import pallas as pl
from jax.experimental.pallas import tpu as pltpu
from jax.experimental.pallas import tpu_sc as plsc

E = 320000
N = 10000
H = 128
R = 6
OUT = 128

NC = 2
NS = 16
NW = NC * NS
EPW = E // NW
EB = 80
NCHUNK = EPW // EB
NG = EB // 16
RPT = 624
ZC = 16
LANES = 16


def _sc_edge_scatter(x, rbft, idx, wt):
    mesh = plsc.VectorSubcoreMesh(core_axis_name="c", subcore_axis_name="s")

    @functools.partial(
        pl.kernel,
        out_type=jax.ShapeDtypeStruct((NC, N, H), jnp.float32),
        mesh=mesh,
        scratch_types=[
            pltpu.VMEM((EB, H), jnp.float32),
            pltpu.VMEM((EB, H), jnp.float32),
            pltpu.VMEM((EB, H), jnp.float32),
            pltpu.VMEM((EB, H), jnp.float32),
            pltpu.VMEM((R, EB), jnp.float32),
            pltpu.VMEM((R, EB), jnp.float32),
            pltpu.VMEM((R, EB), jnp.float32),
            pltpu.VMEM((R, EB), jnp.float32),
            pltpu.VMEM((EB,), jnp.int32),
            pltpu.VMEM((EB,), jnp.int32),
            pltpu.VMEM((EB,), jnp.int32),
            pltpu.VMEM((EB,), jnp.int32),
            pltpu.VMEM((R, H), jnp.float32),
            pltpu.VMEM_SHARED((N, H), jnp.float32),
            pltpu.SemaphoreType.DMA,
            pltpu.SemaphoreType.DMA,
            pltpu.SemaphoreType.DMA,
            pltpu.SemaphoreType.DMA,
            pltpu.SemaphoreType.DMA,
            pltpu.SemaphoreType.DMA,
            pltpu.SemaphoreType.DMA,
            pltpu.SemaphoreType.DMA,
            pltpu.SemaphoreType.DMA,
        ],
    )
    def body(x_hbm, rbft_hbm, i_hbm, wt_hbm, out_hbm,
             xb0, xb1, xb2, xb3, rb0, rb1, rb2, rb3, ib0, ib1, ib2, ib3,
             wtb, acc, lsem0, lsem1, lsem2, lsem3,
             ssem0, ssem1, ssem2, ssem3, zsem):
        cid = lax.axis_index("c")
        sid = lax.axis_index("s")
        wid = sid * NC + cid
        ebase = wid * EPW

        pltpu.async_copy(wt_hbm, wtb, lsem0).wait()

        def zrow(r2, _):
            for k in range(H // LANES):
                xb0[r2, pl.ds(k * LANES, LANES)] = jnp.zeros((LANES,), jnp.float32)
            return 0
        lax.fori_loop(0, EB, zrow, 0)
        row0 = sid * RPT

        def acc_phase(dst_of):
            descs = []
            for t in range(RPT // EB):
                descs.append(dst_of(row0 + t * EB, EB))
            descs.append(dst_of(row0 + (RPT // EB) * EB, RPT % EB))
            for src, dst in descs:
                pltpu.async_copy(src, dst, zsem)

            @pl.when(sid == NS - 1)
            def _():
                s2, d2 = dst_of(NS * RPT, N - NS * RPT)
                pltpu.async_copy(s2, d2, zsem).wait()
            for src, dst in descs:
                pltpu.make_async_copy(src, dst, zsem).wait()

        acc_phase(lambda r0, n: (xb0.at[pl.ds(0, n)], acc.at[pl.ds(r0, n)]))
        plsc.subcore_barrier()

        def start_load(c, xb, rb, ib, sem):
            e0 = ebase + c * EB
            pltpu.async_copy(x_hbm.at[pl.ds(e0, EB), :], xb, sem)
            for r in range(R):
                pltpu.async_copy(rbft_hbm.at[pl.ds(r * E + e0, EB)], rb.at[r], sem)
            pltpu.async_copy(i_hbm.at[pl.ds(e0, EB)], ib, sem)

        def wait_load(xb, rb, ib, sem):
            pltpu.make_async_copy(x_hbm.at[pl.ds(ebase, EB), :], xb, sem).wait()
            for r in range(R):
                pltpu.make_async_copy(rbft_hbm.at[pl.ds(r * E, EB)], rb.at[r], sem).wait()
            pltpu.make_async_copy(i_hbm.at[pl.ds(ebase, EB)], ib, sem).wait()

        splats = [jnp.full((LANES, 1), j, jnp.int32) for j in range(LANES)]
        _gd = lax.GatherDimensionNumbers(
            offset_dims=(), collapsed_slice_dims=(0,), start_index_map=(0,))

        def bcast(v, j):
            return lax.gather(v, splats[j], _gd, (1,),
                              mode=lax.GatherScatterMode.PROMISE_IN_BOUNDS)

        def compute_chunk(xb, rb):
            for quarter in range(4):
                wtv = [[wtb[r, pl.ds((quarter * 2 + k) * LANES, LANES)]
                        for k in range(2)] for r in range(R)]

                def grp(g, _):
                    rv = [rb[r, pl.ds(g * LANES, LANES)] for r in range(R)]
                    for j in range(LANES):
                        row = g * LANES + j
                        cs = [bcast(rv[r], j) for r in range(R)]
                        for k in range(2):
                            kk = quarter * 2 + k
                            p0 = cs[0] * wtv[0][k] + cs[1] * wtv[1][k]
                            p1 = cs[2] * wtv[2][k] + cs[3] * wtv[3][k]
                            p2 = cs[4] * wtv[4][k] + cs[5] * wtv[5][k]
                            w = (p0 + p1) + p2
                            xv = xb[row, pl.ds(kk * LANES, LANES)]
                            xb[row, pl.ds(kk * LANES, LANES)] = w * xv
                    return 0
                lax.fori_loop(0, NG, grp, 0)

        xbs = [xb0, xb1, xb2, xb3]
        rbs = [rb0, rb1, rb2, rb3]
        ibs = [ib0, ib1, ib2, ib3]
        lsems = [lsem0, lsem1, lsem2, lsem3]
        ssems = [ssem0, ssem1, ssem2, ssem3]

        def _always(f):
            f()

        def do_chunk(c, b, first_turn):
            bn = (b + 2) % 4
            wait_load(xbs[b], rbs[b], ibs[b], lsems[b])
            drain = pl.when(c >= 2) if first_turn else _always
            @drain
            def _():
                pltpu.make_async_copy(xbs[bn], acc.at[ibs[bn]], ssems[bn]).wait()
            @pl.when(c + 2 < NCHUNK)
            def _():
                start_load(c + 2, xbs[bn], rbs[bn], ibs[bn], lsems[bn])
            compute_chunk(xbs[b], rbs[b])
            pltpu.async_copy(xbs[b], acc.at[ibs[b]], ssems[b], add=True)

        start_load(0, xb0, rb0, ib0, lsem0)
        start_load(1, xb1, rb1, ib1, lsem1)

        def turn(it, _):
            c0 = it * 4
            for p in range(4):
                do_chunk(c0 + p, p, first_turn=(p < 2))
            return 0
        lax.fori_loop(0, NCHUNK // 4, turn, 0)
        do_chunk(NCHUNK - 1, 0, first_turn=False)

        for c_tail in range(NCHUNK - 2, NCHUNK):
            b = c_tail % 4
            pltpu.make_async_copy(xbs[b], acc.at[ibs[b]], ssems[b]).wait()

        plsc.subcore_barrier()

        acc_phase(lambda r0, n: (acc.at[pl.ds(r0, n)],
                                 out_hbm.at[cid, pl.ds(r0, n), :]))

    return body(x, rbft, idx, wt)


BR = 1000


def _mlp(parts, w1, b1, w2, b2, w3, b3, wout):
    def body(p_ref, w1_ref, b1_ref, w2_ref, b2_ref, w3_ref, b3_ref, wo_ref,
             o_ref):
        h = p_ref[0] + p_ref[1]

        def ff(h, w_ref, b_ref):
            y = lax.dot_general(h, w_ref[...], (((1,), (1,)), ((), ())),
                                precision=lax.Precision.HIGHEST,
                                preferred_element_type=jnp.float32)
            y = y + b_ref[...]
            return y * jax.nn.sigmoid(y)

        h = ff(h, w1_ref, b1_ref)
        h = ff(h, w2_ref, b2_ref)
        h = ff(h, w3_ref, b3_ref)
        o_ref[...] = lax.dot_general(h, wo_ref[...], (((1,), (1,)), ((), ())),
                                     precision=lax.Precision.HIGHEST,
                                     preferred_element_type=jnp.float32)

    wspec = pl.BlockSpec((H, H), lambda b: (0, 0))
    bspec = pl.BlockSpec((1, H), lambda b: (0, 0))
    return pl.pallas_call(
        body,
        grid=(N // BR,),
        in_specs=[
            pl.BlockSpec((NC, BR, H), lambda b: (0, b, 0)),
            wspec, bspec, wspec, bspec, wspec, bspec,
            pl.BlockSpec((OUT, H), lambda b: (0, 0)),
        ],
        out_specs=pl.BlockSpec((BR, OUT), lambda b: (b, 0)),
        out_shape=jax.ShapeDtypeStruct((N, OUT), jnp.float32),
    )(parts, w1, b1, w2, b2, w3, b3, wout)


def kernel(x, rbf, i, num_nodes, W_rbf, W1, b1, W2, b2, W3, b3, W_out):
    del num_nodes
    rbft = rbf.T.reshape(-1)
    idx = i.astype(jnp.int32)
    wt = W_rbf.T
    parts = _sc_edge_scatter(x, rbft, idx, wt)
    return _mlp(parts, W1, b1.reshape(1, H), W2, b2.reshape(1, H),
                W3, b3.reshape(1, H), W_out)
